# baseline (speedup 1.0000x reference)
import functools

import jax
import jax.numpy as jnp
from jax import lax
from jax.experimental import pallas as pl
from jax.experimental.pallas import tpu as pltpu
from jax.experimental.pallas import tpu_sc as plsc

def _sortable(bits):
    flip = jnp.full(bits.shape, 0x7FFFFFFF, jnp.int32)
    return jnp.where(bits >= 0, bits, jnp.bitwise_xor(bits, flip))


def _sc_median_build(rows, n, k):
    nw = 32
    rpw = rows // nw
    mesh = plsc.VectorSubcoreMesh(core_axis_name="c", subcore_axis_name="s")
    cap = n + 16

    @functools.partial(
        pl.kernel,
        mesh=mesh,
        compiler_params=pltpu.CompilerParams(needs_layout_passes=False),
        out_type=jax.ShapeDtypeStruct((rows,), jnp.int32),
        scratch_types=[
            pltpu.VMEM((cap,), jnp.int32),
            pltpu.VMEM((cap,), jnp.int32),
            pltpu.VMEM((4096,), jnp.int32),
            pltpu.VMEM((rpw,), jnp.int32),
        ],
    )
    def sc_kernel(x_hbm, out_hbm, row_v, cand_v, hist_v, res_v):
        wid = lax.axis_index("s") * 2 + lax.axis_index("c")
        lane = lax.iota(jnp.int32, 16)
        ones = jnp.ones((16,), jnp.int32)
        zeros16 = jnp.zeros((16,), jnp.int32)

        def rst(b, _):
            hist_v[pl.ds(b * 16, 16)] = zeros16
            return 0
        lax.fori_loop(0, 256, rst, 0)

        def level(src_ref, dst_ref, m, r, lvl):
            shift = 24 - 8 * lvl
            nv = (m + 15) // 16

            def keybin(i):
                v = src_ref[pl.ds(i * 16, 16)]
                s = _sortable(v) if lvl == 0 else v
                if lvl == 0:
                    bn = (s >> 24) + 128
                else:
                    bn = (s >> shift) & 0xFF
                return s, bn

            def hbody(i, _):
                _, bn = keybin(i)
                msk = None if lvl == 0 else (i * 16 + lane) < m
                plsc.addupdate_scatter(
                    hist_v, [bn * 16 + lane], ones, mask=msk)
                return 0
            if lvl == 0:
                lax.fori_loop(0, n // 16, hbody, 0, unroll=8)
            else:
                lax.fori_loop(0, nv, hbody, 0)

            def fbody(i, carry):
                acc, t, above = carry
                b = 255 - i
                cb = jnp.sum(hist_v[pl.ds(b * 16, 16)])
                hist_v[pl.ds(b * 16, 16)] = zeros16
                newacc = acc + cb
                hit = (acc < r) & (newacc >= r)
                return (newacc,
                        jnp.where(hit, b, t),
                        jnp.where(hit, acc, above))
            _, t, above = lax.fori_loop(
                0, 256, fbody,
                (jnp.int32(0), jnp.int32(0), jnp.int32(0)))
            r2 = r - above

            if lvl < 3:
                def cbody(i, off):
                    s, bn = keybin(i)
                    sel = bn == t
                    if lvl != 0:
                        sel = sel & ((i * 16 + lane) < m)
                    plsc.store_compressed(
                        dst_ref.at[pl.ds(off, 16)], s, mask=sel)
                    return off + jnp.sum(sel.astype(jnp.int32))
                if lvl == 0:
                    m2 = lax.fori_loop(0, n // 16, cbody, jnp.int32(0),
                                       unroll=8)
                else:
                    m2 = lax.fori_loop(0, nv, cbody, jnp.int32(0))
            else:
                m2 = m
            return m2, r2, t

        def do_row(j, _):
            row = wid * rpw + j
            pltpu.sync_copy(x_hbm.at[row], row_v.at[pl.ds(0, n)])
            m1, r1, t0 = level(row_v, cand_v, jnp.int32(n), jnp.int32(k), 0)
            m2, r2, t1 = level(cand_v, row_v, m1, r1, 1)
            m3, r3, t2 = level(row_v, cand_v, m2, r2, 2)
            _, _, t3 = level(cand_v, row_v, m3, r3, 3)
            s_ans = ((t0 - 128) << 24) | (t1 << 16) | (t2 << 8) | t3
            plsc.store_scatter(
                res_v, [jnp.broadcast_to(j, (16,))],
                jnp.broadcast_to(s_ans, (16,)), mask=(lane == 0))
            return 0
        lax.fori_loop(0, rpw, do_row, 0)
        pltpu.sync_copy(res_v, out_hbm.at[pl.ds(wid * rpw, rpw)])

    return sc_kernel


def _dense_body(s_ref, w_ref, b_ref, o_ref):
    med = lax.bitcast_convert_type(_sortable(s_ref[...]), jnp.float32)
    o_ref[...] = (
        jnp.dot(med, w_ref[...], preferred_element_type=jnp.float32)
        + b_ref[...]
    )


def kernel(x, W, b):
    B, C, H, Wsp = x.shape
    n = H * Wsp
    k = n // 2
    rows = B * C
    xi = lax.bitcast_convert_type(x.reshape(rows, n), jnp.int32)

    s_med = _sc_median_build(rows, n, k)(xi)

    out = pl.pallas_call(
        _dense_body,
        out_shape=jax.ShapeDtypeStruct((B, W.shape[0]), jnp.float32),
    )(s_med.reshape(B, C), W.T, b.reshape(1, -1))
    return out


# speedup vs baseline: 2.7636x; 2.7636x over previous
import functools

import jax
import jax.numpy as jnp
from jax import lax
from jax.experimental import pallas as pl
from jax.experimental.pallas import tpu as pltpu
from jax.experimental.pallas import tpu_sc as plsc

def _sortable(bits):
    flip = jnp.full(bits.shape, 0x7FFFFFFF, jnp.int32)
    return jnp.where(bits >= 0, bits, jnp.bitwise_xor(bits, flip))


def _sc_median_build(rows, n, k):
    nw = 32
    rpw = rows // nw
    mesh = plsc.VectorSubcoreMesh(core_axis_name="c", subcore_axis_name="s")
    cap = n + 16

    @functools.partial(
        pl.kernel,
        mesh=mesh,
        compiler_params=pltpu.CompilerParams(needs_layout_passes=False),
        out_type=jax.ShapeDtypeStruct((rows,), jnp.int32),
        scratch_types=[
            pltpu.VMEM((cap,), jnp.int32),
            pltpu.VMEM((cap,), jnp.int32),
            pltpu.VMEM((4096,), jnp.int32),
            pltpu.VMEM((rpw,), jnp.int32),
        ],
    )
    def sc_kernel(x_hbm, out_hbm, row_v, cand_v, hist_v, res_v):
        wid = lax.axis_index("s") * 2 + lax.axis_index("c")
        lane = lax.iota(jnp.int32, 16)
        ones = jnp.ones((16,), jnp.int32)
        zeros16 = jnp.zeros((16,), jnp.int32)

        def rst(b, _):
            hist_v[pl.ds(b * 16, 16)] = zeros16
            return 0
        lax.fori_loop(0, 256, rst, 0)

        def level(src_ref, dst_ref, m, r, lvl):
            shift = 24 - 8 * lvl
            nv = (m + 15) // 16

            def keybin(i):
                v = src_ref[pl.ds(i * 16, 16)]
                s = _sortable(v) if lvl == 0 else v
                if lvl == 0:
                    bn = (s >> 24) + 128
                else:
                    bn = (s >> shift) & 0xFF
                return s, bn

            @plsc.parallel_loop(0, n // 16 if lvl == 0 else nv,
                                unroll=8 if lvl == 0 else 1)
            def _(i):
                _, bn = keybin(i)
                msk = None if lvl == 0 else (i * 16 + lane) < m
                plsc.addupdate_scatter(
                    hist_v, [bn * 16 + lane], ones, mask=msk)

            @plsc.parallel_loop(
                0, 256, unroll=4,
                carry=(jnp.int32(0), jnp.int32(0), jnp.int32(0)))
            def _find(i, carry):
                acc, t, above = carry
                b = 255 - i
                cb = jnp.sum(hist_v[pl.ds(b * 16, 16)])
                hist_v[pl.ds(b * 16, 16)] = zeros16
                newacc = acc + cb
                hit = (acc < r) & (newacc >= r)
                return (newacc,
                        jnp.where(hit, b, t),
                        jnp.where(hit, acc, above))
            _, t, above = _find
            r2 = r - above

            if lvl < 3:
                @plsc.parallel_loop(0, n // 16 if lvl == 0 else nv,
                                    unroll=8 if lvl == 0 else 1,
                                    carry=jnp.int32(0))
                def m2(i, off):
                    s, bn = keybin(i)
                    sel = bn == t
                    if lvl != 0:
                        sel = sel & ((i * 16 + lane) < m)
                    plsc.store_compressed(
                        dst_ref.at[pl.ds(off, 16)], s, mask=sel)
                    return off + jnp.sum(sel.astype(jnp.int32))
            else:
                m2 = m
            return m2, r2, t

        def do_row(j, _):
            row = wid * rpw + j
            pltpu.sync_copy(x_hbm.at[row], row_v.at[pl.ds(0, n)])
            m1, r1, t0 = level(row_v, cand_v, jnp.int32(n), jnp.int32(k), 0)
            m2, r2, t1 = level(cand_v, row_v, m1, r1, 1)
            m3, r3, t2 = level(row_v, cand_v, m2, r2, 2)
            _, _, t3 = level(cand_v, row_v, m3, r3, 3)
            s_ans = ((t0 - 128) << 24) | (t1 << 16) | (t2 << 8) | t3
            plsc.store_scatter(
                res_v, [jnp.broadcast_to(j, (16,))],
                jnp.broadcast_to(s_ans, (16,)), mask=(lane == 0))
            return 0
        lax.fori_loop(0, rpw, do_row, 0)
        pltpu.sync_copy(res_v, out_hbm.at[pl.ds(wid * rpw, rpw)])

    return sc_kernel


def _dense_body(s_ref, w_ref, b_ref, o_ref):
    med = lax.bitcast_convert_type(_sortable(s_ref[...]), jnp.float32)
    o_ref[...] = (
        jnp.dot(med, w_ref[...], preferred_element_type=jnp.float32)
        + b_ref[...]
    )


def kernel(x, W, b):
    B, C, H, Wsp = x.shape
    n = H * Wsp
    k = n // 2
    rows = B * C
    xi = lax.bitcast_convert_type(x.reshape(rows, n), jnp.int32)

    s_med = _sc_median_build(rows, n, k)(xi)

    out = pl.pallas_call(
        _dense_body,
        out_shape=jax.ShapeDtypeStruct((B, W.shape[0]), jnp.float32),
    )(s_med.reshape(B, C), W.T, b.reshape(1, -1))
    return out
